# baseline (speedup 1.0000x reference)
import functools

import jax
import jax.numpy as jnp
from jax import lax
from jax.experimental import pallas as pl
from jax.experimental.pallas import tpu as pltpu
from jax.experimental.pallas import tpu_sc as plsc

_NC, _NS = 2, 16
_NW = _NC * _NS
_C = 96
_CH = 108
_SCH = 81
_E2 = _NW * _CH * _C
_NP = 10240


def _sc_mesh():
    return plsc.VectorSubcoreMesh(
        core_axis_name="c", subcore_axis_name="s",
        num_cores=_NC, num_subcores=_NS)


def _make_gather(F):
    per_w = _CH * _C

    @functools.partial(
        pl.kernel,
        out_type=(jax.ShapeDtypeStruct((_E2, F), jnp.float32),
                  jax.ShapeDtypeStruct((_E2, F), jnp.float32),
                  jax.ShapeDtypeStruct((_E2, 8), jnp.float32)),
        mesh=_sc_mesh(),
        scratch_types=[
            pltpu.VMEM((_CH, _C), jnp.int32),
            pltpu.VMEM((_CH, _C), jnp.int32),
            pltpu.VMEM((_C, F), jnp.float32),
            pltpu.VMEM((_C, F), jnp.float32),
            pltpu.VMEM((_C, F), jnp.float32),
            pltpu.VMEM((_C, F), jnp.float32),
            pltpu.VMEM((_C, 8), jnp.float32),
            pltpu.VMEM((_NP,), jnp.float32),
            pltpu.VMEM((_NP,), jnp.float32),
            pltpu.VMEM((_NP,), jnp.float32),
            pltpu.SemaphoreType.DMA,
            pltpu.SemaphoreType.DMA,
            pltpu.SemaphoreType.DMA,
            pltpu.SemaphoreType.DMA,
        ],
        compiler_params=pltpu.CompilerParams(needs_layout_passes=False),
    )
    def gather_k(x_hbm, px_hbm, py_hbm, pz_hbm, row_hbm, col_hbm,
                 outr_hbm, outc_hbm, de_hbm,
                 idxr, idxc, rb0, cb0, rb1, cb1, db,
                 pxv, pyv, pzv, gsem, gsem2, wsem, dsem):
        wid = lax.axis_index("s") * _NC + lax.axis_index("c")
        base = wid * per_w
        pltpu.sync_copy(row_hbm.at[wid], idxr)
        pltpu.sync_copy(col_hbm.at[wid], idxc)
        pltpu.sync_copy(px_hbm, pxv)
        pltpu.sync_copy(py_hbm, pyv)
        pltpu.sync_copy(pz_hbm, pzv)
        bufs = ((rb0, cb0), (rb1, cb1))
        lane = lax.iota(jnp.int32, 16)

        def half(i, b, drain_rc, drain_d):
            rb, cb = bufs[b]
            if drain_rc:
                pltpu.make_async_copy(rb, outr_hbm.at[pl.ds(0, _C)], wsem).wait()
                pltpu.make_async_copy(cb, outc_hbm.at[pl.ds(0, _C)], wsem).wait()
            gr = pltpu.async_copy(x_hbm.at[idxr.at[i]], rb, gsem)
            gc = pltpu.async_copy(x_hbm.at[idxc.at[i]], cb, gsem2)
            if drain_d:
                pltpu.make_async_copy(db, de_hbm.at[pl.ds(0, _C)], dsem).wait()
            for j in range(_C // 16):
                k16 = j * 16
                ir = idxr[i, pl.ds(k16, 16)]
                ic = idxc[i, pl.ds(k16, 16)]
                ridx = k16 + lane
                for comp, pref in enumerate((pxv, pyv, pzv)):
                    dv = (plsc.load_gather(pref, [ir])
                          - plsc.load_gather(pref, [ic]))
                    cidx = jnp.full((16,), comp, jnp.int32)
                    plsc.store_scatter(db, [ridx, cidx], dv)
            eb = base + i * _C
            pltpu.async_copy(db, de_hbm.at[pl.ds(eb, _C)], dsem)
            gr.wait()
            gc.wait()
            pltpu.async_copy(rb, outr_hbm.at[pl.ds(eb, _C)], wsem)
            pltpu.async_copy(cb, outc_hbm.at[pl.ds(eb, _C)], wsem)

        def body(i2, _):
            @pl.when(i2 > 0)
            def _():
                half(2 * i2, 0, True, True)
                half(2 * i2 + 1, 1, True, True)
            return 0

        half(0, 0, False, False)
        half(1, 1, False, True)
        lax.fori_loop(1, _CH // 2, body, 0)
        for b in range(2):
            rb, cb = bufs[b]
            pltpu.make_async_copy(rb, outr_hbm.at[pl.ds(0, _C)], wsem).wait()
            pltpu.make_async_copy(cb, outc_hbm.at[pl.ds(0, _C)], wsem).wait()
        pltpu.make_async_copy(db, de_hbm.at[pl.ds(0, _C)], dsem).wait()

    return gather_k


def _make_scatter():
    acc_rows = _NP * 8 // 128

    @functools.partial(
        pl.kernel,
        out_type=jax.ShapeDtypeStruct((_NW, acc_rows, 128), jnp.float32),
        mesh=_sc_mesh(),
        scratch_types=[
            pltpu.VMEM((_SCH, 128), jnp.int32),
            pltpu.VMEM((8, 128), jnp.float32),
            pltpu.VMEM((acc_rows, 128), jnp.float32),
        ],
        compiler_params=pltpu.CompilerParams(needs_layout_passes=False),
    )
    def scatter_k(ve_hbm, col_hbm, zeros_hbm, out_hbm, idxv, vbuf, acc):
        wid = lax.axis_index("s") * _NC + lax.axis_index("c")
        pltpu.sync_copy(zeros_hbm, acc)
        pltpu.sync_copy(col_hbm.at[wid], idxv)
        cbase = wid * _SCH * 128

        def body(i, _):
            pltpu.sync_copy(ve_hbm.at[:, pl.ds(cbase + i * 128, 128)], vbuf)
            for j in range(8):
                ids = idxv[i, pl.ds(j * 16, 16)]
                for comp in (0, 1, 2, 4, 5, 6):
                    vals = vbuf[comp, pl.ds(j * 16, 16)]
                    tgt = ids * 8 + comp
                    plsc.addupdate_scatter(
                        acc, [lax.shift_right_logical(tgt, 7),
                              lax.bitwise_and(tgt, 127)], vals)
            return 0

        lax.fori_loop(0, _SCH, body, 0)
        pltpu.sync_copy(acc, out_hbm.at[wid])

    return scatter_k


def _edge_body(xr_ref, xc_ref, de_ref, wr_ref, wc_ref, w2_ref, aux_ref,
               out_ref):
    bf = jnp.bfloat16
    f32 = jnp.float32
    xr = xr_ref[...].astype(bf)
    xc = xc_ref[...].astype(bf)
    h = (jnp.dot(xr, wr_ref[...], preferred_element_type=f32)
         + jnp.dot(xc, wc_ref[...], preferred_element_type=f32))
    d = de_ref[...][:, 0:3]
    dist2 = jnp.sum(d * d, axis=1, keepdims=True)
    dist = jnp.sqrt(dist2)
    dterm = (dist.astype(bf).astype(f32)
             * aux_ref[0:1, :].astype(bf).astype(f32))
    h = h + dterm + aux_ref[1:2, :]
    s = h * jax.nn.sigmoid(h)
    mm = jnp.dot(s.astype(bf), w2_ref[...], preferred_element_type=f32)
    m1 = mm[:, 0:1] + aux_ref[3:4, 0:1]
    m2 = mm[:, 1:2] + aux_ref[3:4, 1:2]
    r = dist / 4.5
    r2 = r * r
    r4 = r2 * r2
    r5 = r4 * r
    r6 = r4 * r2
    r7 = r4 * r2 * r
    coe = 1.0 - 21.0 * r5 + 35.0 * r6 - 15.0 * r7
    nv = d / (dist + 1e-8)
    be = xr.shape[0]
    z = jnp.zeros((be, 1), f32)
    vec1 = nv * coe * m1
    vec2 = nv * coe * m2
    packed = jnp.concatenate([vec1, z, vec2, z], axis=1)
    out_ref[...] = packed.T


def _finish_body(p_ref, out_ref):
    v = jnp.sum(p_ref[...], axis=0)
    v1 = v[:, 0:3]
    v2 = v[:, 4:7]
    n1 = v1 / (jnp.sqrt(jnp.sum(v1 * v1, axis=1, keepdims=True)) + 1e-8)
    dot = jnp.sum(n1 * v2, axis=1, keepdims=True)
    n2p = v2 - dot * n1
    n2 = n2p / (jnp.sqrt(jnp.sum(n2p * n2p, axis=1, keepdims=True)) + 1e-8)
    a0, a1, a2 = n1[:, 0:1], n1[:, 1:2], n1[:, 2:3]
    b0, b1, b2 = n2[:, 0:1], n2[:, 1:2], n2[:, 2:3]
    n3 = jnp.concatenate(
        [a1 * b2 - a2 * b1, a2 * b0 - a0 * b2, a0 * b1 - a1 * b0], axis=1)
    out_ref[...] = jnp.concatenate([n1, n2, n3], axis=1)


def kernel(x, pos, W1a, b1a, W1b, b1b, W2a, b2a, W2b, b2b, edge_index):
    N = x.shape[0]
    E = edge_index.shape[1]
    F = x.shape[1]
    H = W1a.shape[1]

    xt = jnp.zeros((_NP, F), jnp.float32).at[:N].set(x)
    pp = jnp.zeros((3, _NP), jnp.float32).at[:, :N].set(pos.T)
    row = (jnp.zeros((_E2,), jnp.int32)
           .at[:E].set(edge_index[0].astype(jnp.int32)).reshape(_NW, _CH, _C))
    colf = (jnp.full((_E2,), N, jnp.int32)
            .at[:E].set(edge_index[1].astype(jnp.int32)))
    col = colf.reshape(_NW, _CH, _C)
    cols = colf.reshape(_NW, _SCH, 128)

    Wr = jnp.concatenate([W1a[0:F], W2a[0:F]], axis=1).astype(jnp.bfloat16)
    Wc = jnp.concatenate([W1a[F:2 * F], W2a[F:2 * F]],
                         axis=1).astype(jnp.bfloat16)
    W2 = jnp.zeros((2 * H, 128), jnp.float32)
    W2 = W2.at[0:H, 0].set(W1b[:, 0]).at[H:2 * H, 1].set(W2b[:, 0])
    W2 = W2.astype(jnp.bfloat16)

    aux = jnp.zeros((8, 2 * H), jnp.float32)
    aux = aux.at[0, 0:H].set(W1a[2 * F]).at[0, H:].set(W2a[2 * F])
    aux = aux.at[1, 0:H].set(b1a).at[1, H:].set(b2a)
    aux = aux.at[3, 0].set(b1b[0]).at[3, 1].set(b2b[0])

    xer, xec, de = _make_gather(F)(xt, pp[0], pp[1], pp[2], row, col)

    BE = 1024
    ve = pl.pallas_call(
        _edge_body,
        grid=(_E2 // BE,),
        in_specs=[
            pl.BlockSpec((BE, F), lambda i: (i, 0)),
            pl.BlockSpec((BE, F), lambda i: (i, 0)),
            pl.BlockSpec((BE, 8), lambda i: (i, 0)),
            pl.BlockSpec((F, 2 * H), lambda i: (0, 0)),
            pl.BlockSpec((F, 2 * H), lambda i: (0, 0)),
            pl.BlockSpec((2 * H, 128), lambda i: (0, 0)),
            pl.BlockSpec((8, 2 * H), lambda i: (0, 0)),
        ],
        out_specs=pl.BlockSpec((8, BE), lambda i: (0, i)),
        out_shape=jax.ShapeDtypeStruct((8, _E2), jnp.float32),
    )(xer, xec, de, Wr, Wc, W2, aux)

    zacc = jnp.zeros((_NP * 8 // 128, 128), jnp.float32)
    partials = _make_scatter()(ve, cols, zacc)
    partials = partials.reshape(_NW, _NP, 8)

    NB = _NP // 8
    out9 = pl.pallas_call(
        _finish_body,
        grid=(_NP // NB,),
        in_specs=[pl.BlockSpec((_NW, NB, 8), lambda i: (0, i, 0))],
        out_specs=pl.BlockSpec((NB, 9), lambda i: (i, 0)),
        out_shape=jax.ShapeDtypeStruct((_NP, 9), jnp.float32),
    )(partials)

    return out9[:N].reshape(N, 3, 3)


# speedup vs baseline: 1.1922x; 1.1922x over previous
import functools

import jax
import jax.numpy as jnp
from jax import lax
from jax.experimental import pallas as pl
from jax.experimental.pallas import tpu as pltpu
from jax.experimental.pallas import tpu_sc as plsc

_NC, _NS = 2, 16
_NW = _NC * _NS
_C = 96
_CH = 106
_E2 = _NW * _CH * _C
_NP = 10240


def _sc_mesh():
    return plsc.VectorSubcoreMesh(
        core_axis_name="c", subcore_axis_name="s",
        num_cores=_NC, num_subcores=_NS)


def _make_gather(F):
    per_w = _CH * _C

    @functools.partial(
        pl.kernel,
        out_type=(jax.ShapeDtypeStruct((_E2, F), jnp.float32),
                  jax.ShapeDtypeStruct((_E2, F), jnp.float32),
                  jax.ShapeDtypeStruct((_E2, 8), jnp.float32)),
        mesh=_sc_mesh(),
        scratch_types=[
            pltpu.VMEM((_CH, _C), jnp.int32),
            pltpu.VMEM((_CH, _C), jnp.int32),
            pltpu.VMEM((_C, F), jnp.float32),
            pltpu.VMEM((_C, F), jnp.float32),
            pltpu.VMEM((_C, F), jnp.float32),
            pltpu.VMEM((_C, F), jnp.float32),
            pltpu.VMEM((_C, 8), jnp.float32),
            pltpu.VMEM((_NP,), jnp.float32),
            pltpu.VMEM((_NP,), jnp.float32),
            pltpu.VMEM((_NP,), jnp.float32),
            pltpu.SemaphoreType.DMA,
            pltpu.SemaphoreType.DMA,
            pltpu.SemaphoreType.DMA,
            pltpu.SemaphoreType.DMA,
        ],
        compiler_params=pltpu.CompilerParams(needs_layout_passes=False),
    )
    def gather_k(x_hbm, px_hbm, py_hbm, pz_hbm, row_hbm, col_hbm,
                 outr_hbm, outc_hbm, de_hbm,
                 idxr, idxc, rb0, cb0, rb1, cb1, db,
                 pxv, pyv, pzv, gsem, gsem2, wsem, dsem):
        wid = lax.axis_index("s") * _NC + lax.axis_index("c")
        base = wid * per_w
        pltpu.sync_copy(row_hbm.at[wid], idxr)
        pltpu.sync_copy(col_hbm.at[wid], idxc)
        pltpu.sync_copy(px_hbm, pxv)
        pltpu.sync_copy(py_hbm, pyv)
        pltpu.sync_copy(pz_hbm, pzv)
        bufs = ((rb0, cb0), (rb1, cb1))
        lane = lax.iota(jnp.int32, 16)

        def half(i, b, drain_rc, drain_d):
            rb, cb = bufs[b]
            if drain_rc:
                pltpu.make_async_copy(rb, outr_hbm.at[pl.ds(0, _C)], wsem).wait()
                pltpu.make_async_copy(cb, outc_hbm.at[pl.ds(0, _C)], wsem).wait()
            gr = pltpu.async_copy(x_hbm.at[idxr.at[i]], rb, gsem)
            gc = pltpu.async_copy(x_hbm.at[idxc.at[i]], cb, gsem2)
            if drain_d:
                pltpu.make_async_copy(db, de_hbm.at[pl.ds(0, _C)], dsem).wait()
            for j in range(_C // 16):
                k16 = j * 16
                ir = idxr[i, pl.ds(k16, 16)]
                ic = idxc[i, pl.ds(k16, 16)]
                ridx = k16 + lane
                for comp, pref in enumerate((pxv, pyv, pzv)):
                    dv = (plsc.load_gather(pref, [ir])
                          - plsc.load_gather(pref, [ic]))
                    cidx = jnp.full((16,), comp, jnp.int32)
                    plsc.store_scatter(db, [ridx, cidx], dv)
            eb = base + i * _C
            pltpu.async_copy(db, de_hbm.at[pl.ds(eb, _C)], dsem)
            gr.wait()
            gc.wait()
            pltpu.async_copy(rb, outr_hbm.at[pl.ds(eb, _C)], wsem)
            pltpu.async_copy(cb, outc_hbm.at[pl.ds(eb, _C)], wsem)

        def body(i2, _):
            @pl.when(i2 > 0)
            def _():
                half(2 * i2, 0, True, True)
                half(2 * i2 + 1, 1, True, True)
            return 0

        half(0, 0, False, False)
        half(1, 1, False, True)
        lax.fori_loop(1, _CH // 2, body, 0)
        for b in range(2):
            rb, cb = bufs[b]
            pltpu.make_async_copy(rb, outr_hbm.at[pl.ds(0, _C)], wsem).wait()
            pltpu.make_async_copy(cb, outc_hbm.at[pl.ds(0, _C)], wsem).wait()
        pltpu.make_async_copy(db, de_hbm.at[pl.ds(0, _C)], dsem).wait()

    return gather_k


def _make_scatter():
    acc_rows = _NP * 8 // 128

    @functools.partial(
        pl.kernel,
        out_type=jax.ShapeDtypeStruct((_NW, acc_rows, 128), jnp.float32),
        mesh=_sc_mesh(),
        scratch_types=[
            pltpu.VMEM((_CH, _C), jnp.int32),
            pltpu.VMEM((_C, 8), jnp.float32),
            pltpu.VMEM((acc_rows, 128), jnp.float32),
        ],
        compiler_params=pltpu.CompilerParams(needs_layout_passes=False),
    )
    def scatter_k(ve_hbm, col_hbm, zeros_hbm, out_hbm, idxv, vbuf, acc):
        wid = lax.axis_index("s") * _NC + lax.axis_index("c")
        pltpu.sync_copy(zeros_hbm, acc)
        pltpu.sync_copy(col_hbm.at[wid], idxv)
        vbase = wid * _CH * _C
        lane = lax.iota(jnp.int32, 16)

        def body(i, _):
            pltpu.sync_copy(ve_hbm.at[pl.ds(vbase + i * _C, _C)], vbuf)
            for j in range(_C // 16):
                ids = idxv[i, pl.ds(j * 16, 16)]
                rows = j * 16 + lane
                for comp in (0, 1, 2, 4, 5, 6):
                    vals = plsc.load_gather(
                        vbuf, [rows, jnp.full((16,), comp, jnp.int32)])
                    tgt = ids * 8 + comp
                    plsc.addupdate_scatter(
                        acc, [lax.shift_right_logical(tgt, 7),
                              lax.bitwise_and(tgt, 127)], vals)
            return 0

        lax.fori_loop(0, _CH, body, 0)
        pltpu.sync_copy(acc, out_hbm.at[wid])

    return scatter_k


def _edge_body(xr_ref, xc_ref, de_ref, wr_ref, wc_ref, w2_ref, aux_ref,
               out_ref):
    bf = jnp.bfloat16
    f32 = jnp.float32
    xr = xr_ref[...].astype(bf)
    xc = xc_ref[...].astype(bf)
    h = (jnp.dot(xr, wr_ref[...], preferred_element_type=f32)
         + jnp.dot(xc, wc_ref[...], preferred_element_type=f32))
    d = de_ref[...][:, 0:3]
    dist2 = jnp.sum(d * d, axis=1, keepdims=True)
    dist = jnp.sqrt(dist2)
    dterm = (dist.astype(bf).astype(f32)
             * aux_ref[0:1, :].astype(bf).astype(f32))
    h = h + dterm + aux_ref[1:2, :]
    s = h * jax.nn.sigmoid(h)
    mm = jnp.dot(s.astype(bf), w2_ref[...], preferred_element_type=f32)
    m1 = mm[:, 0:1] + aux_ref[3:4, 0:1]
    m2 = mm[:, 1:2] + aux_ref[3:4, 1:2]
    r = dist / 4.5
    r2 = r * r
    r4 = r2 * r2
    r5 = r4 * r
    r6 = r4 * r2
    r7 = r4 * r2 * r
    coe = 1.0 - 21.0 * r5 + 35.0 * r6 - 15.0 * r7
    nv = d / (dist + 1e-8)
    be = xr.shape[0]
    z = jnp.zeros((be, 1), f32)
    vec1 = nv * coe * m1
    vec2 = nv * coe * m2
    out_ref[...] = jnp.concatenate([vec1, z, vec2, z], axis=1)


def _finish_body(p_ref, out_ref):
    v = jnp.sum(p_ref[...], axis=0)
    v1 = v[:, 0:3]
    v2 = v[:, 4:7]
    n1 = v1 / (jnp.sqrt(jnp.sum(v1 * v1, axis=1, keepdims=True)) + 1e-8)
    dot = jnp.sum(n1 * v2, axis=1, keepdims=True)
    n2p = v2 - dot * n1
    n2 = n2p / (jnp.sqrt(jnp.sum(n2p * n2p, axis=1, keepdims=True)) + 1e-8)
    a0, a1, a2 = n1[:, 0:1], n1[:, 1:2], n1[:, 2:3]
    b0, b1, b2 = n2[:, 0:1], n2[:, 1:2], n2[:, 2:3]
    n3 = jnp.concatenate(
        [a1 * b2 - a2 * b1, a2 * b0 - a0 * b2, a0 * b1 - a1 * b0], axis=1)
    out_ref[...] = jnp.concatenate([n1, n2, n3], axis=1)


def kernel(x, pos, W1a, b1a, W1b, b1b, W2a, b2a, W2b, b2b, edge_index):
    N = x.shape[0]
    E = edge_index.shape[1]
    F = x.shape[1]
    H = W1a.shape[1]

    xt = jnp.zeros((_NP, F), jnp.float32).at[:N].set(x)
    pp = jnp.zeros((3, _NP), jnp.float32).at[:, :N].set(pos.T)
    row = (jnp.zeros((_E2,), jnp.int32)
           .at[:E].set(edge_index[0].astype(jnp.int32)).reshape(_NW, _CH, _C))
    col = (jnp.full((_E2,), N, jnp.int32)
           .at[:E].set(edge_index[1].astype(jnp.int32)).reshape(_NW, _CH, _C))

    Wr = jnp.concatenate([W1a[0:F], W2a[0:F]], axis=1).astype(jnp.bfloat16)
    Wc = jnp.concatenate([W1a[F:2 * F], W2a[F:2 * F]],
                         axis=1).astype(jnp.bfloat16)
    W2 = jnp.zeros((2 * H, 128), jnp.float32)
    W2 = W2.at[0:H, 0].set(W1b[:, 0]).at[H:2 * H, 1].set(W2b[:, 0])
    W2 = W2.astype(jnp.bfloat16)

    aux = jnp.zeros((8, 2 * H), jnp.float32)
    aux = aux.at[0, 0:H].set(W1a[2 * F]).at[0, H:].set(W2a[2 * F])
    aux = aux.at[1, 0:H].set(b1a).at[1, H:].set(b2a)
    aux = aux.at[3, 0].set(b1b[0]).at[3, 1].set(b2b[0])

    xer, xec, de = _make_gather(F)(xt, pp[0], pp[1], pp[2], row, col)

    BE = 1024
    ve = pl.pallas_call(
        _edge_body,
        grid=(_E2 // BE,),
        in_specs=[
            pl.BlockSpec((BE, F), lambda i: (i, 0)),
            pl.BlockSpec((BE, F), lambda i: (i, 0)),
            pl.BlockSpec((BE, 8), lambda i: (i, 0)),
            pl.BlockSpec((F, 2 * H), lambda i: (0, 0)),
            pl.BlockSpec((F, 2 * H), lambda i: (0, 0)),
            pl.BlockSpec((2 * H, 128), lambda i: (0, 0)),
            pl.BlockSpec((8, 2 * H), lambda i: (0, 0)),
        ],
        out_specs=pl.BlockSpec((BE, 8), lambda i: (i, 0)),
        out_shape=jax.ShapeDtypeStruct((_E2, 8), jnp.float32),
    )(xer, xec, de, Wr, Wc, W2, aux)

    zacc = jnp.zeros((_NP * 8 // 128, 128), jnp.float32)
    partials = _make_scatter()(ve, col, zacc)
    partials = partials.reshape(_NW, _NP, 8)

    NB = _NP // 8
    out9 = pl.pallas_call(
        _finish_body,
        grid=(_NP // NB,),
        in_specs=[pl.BlockSpec((_NW, NB, 8), lambda i: (0, i, 0))],
        out_specs=pl.BlockSpec((NB, 9), lambda i: (i, 0)),
        out_shape=jax.ShapeDtypeStruct((_NP, 9), jnp.float32),
    )(partials)

    return out9[:N].reshape(N, 3, 3)


# speedup vs baseline: 1.2154x; 1.0194x over previous
import functools

import jax
import jax.numpy as jnp
from jax import lax
from jax.experimental import pallas as pl
from jax.experimental.pallas import tpu as pltpu
from jax.experimental.pallas import tpu_sc as plsc

_NC, _NS = 2, 16
_NW = _NC * _NS
_C = 96
_CH = 106
_E2 = _NW * _CH * _C
_NP = 10240


def _sc_mesh():
    return plsc.VectorSubcoreMesh(
        core_axis_name="c", subcore_axis_name="s",
        num_cores=_NC, num_subcores=_NS)


def _make_gather(F):
    per_w = _CH * _C

    @functools.partial(
        pl.kernel,
        out_type=(jax.ShapeDtypeStruct((_E2, F), jnp.float32),
                  jax.ShapeDtypeStruct((_E2, F), jnp.float32),
                  jax.ShapeDtypeStruct((_E2, 8), jnp.float32)),
        mesh=_sc_mesh(),
        scratch_types=[
            pltpu.VMEM((_CH, _C), jnp.int32),
            pltpu.VMEM((_CH, _C), jnp.int32),
            pltpu.VMEM((_C, F), jnp.float32),
            pltpu.VMEM((_C, F), jnp.float32),
            pltpu.VMEM((_C, F), jnp.float32),
            pltpu.VMEM((_C, F), jnp.float32),
            pltpu.VMEM((_C, 8), jnp.float32),
            pltpu.VMEM((_NP,), jnp.float32),
            pltpu.VMEM((_NP,), jnp.float32),
            pltpu.VMEM((_NP,), jnp.float32),
            pltpu.SemaphoreType.DMA,
            pltpu.SemaphoreType.DMA,
            pltpu.SemaphoreType.DMA,
            pltpu.SemaphoreType.DMA,
        ],
        compiler_params=pltpu.CompilerParams(needs_layout_passes=False),
    )
    def gather_k(x_hbm, px_hbm, py_hbm, pz_hbm, row_hbm, col_hbm,
                 outr_hbm, outc_hbm, de_hbm,
                 idxr, idxc, rb0, cb0, rb1, cb1, db,
                 pxv, pyv, pzv, gsem, gsem2, wsem, dsem):
        wid = lax.axis_index("s") * _NC + lax.axis_index("c")
        base = wid * per_w
        pltpu.sync_copy(row_hbm.at[wid], idxr)
        pltpu.sync_copy(col_hbm.at[wid], idxc)
        pltpu.sync_copy(px_hbm, pxv)
        pltpu.sync_copy(py_hbm, pyv)
        pltpu.sync_copy(pz_hbm, pzv)
        bufs = ((rb0, cb0), (rb1, cb1))
        lane = lax.iota(jnp.int32, 16)

        def half(i, b, drain_rc, drain_d):
            rb, cb = bufs[b]
            if drain_rc:
                pltpu.make_async_copy(rb, outr_hbm.at[pl.ds(0, _C)], wsem).wait()
                pltpu.make_async_copy(cb, outc_hbm.at[pl.ds(0, _C)], wsem).wait()
            gr = pltpu.async_copy(x_hbm.at[idxr.at[i]], rb, gsem)
            gc = pltpu.async_copy(x_hbm.at[idxc.at[i]], cb, gsem2)
            if drain_d:
                pltpu.make_async_copy(db, de_hbm.at[pl.ds(0, _C)], dsem).wait()
            for j in range(_C // 16):
                k16 = j * 16
                ir = idxr[i, pl.ds(k16, 16)]
                ic = idxc[i, pl.ds(k16, 16)]
                ridx = k16 + lane
                for comp, pref in enumerate((pxv, pyv, pzv)):
                    dv = (plsc.load_gather(pref, [ir])
                          - plsc.load_gather(pref, [ic]))
                    cidx = jnp.full((16,), comp, jnp.int32)
                    plsc.store_scatter(db, [ridx, cidx], dv)
            eb = base + i * _C
            pltpu.async_copy(db, de_hbm.at[pl.ds(eb, _C)], dsem)
            gr.wait()
            gc.wait()
            pltpu.async_copy(rb, outr_hbm.at[pl.ds(eb, _C)], wsem)
            pltpu.async_copy(cb, outc_hbm.at[pl.ds(eb, _C)], wsem)

        def body(i2, _):
            @pl.when(i2 > 0)
            def _():
                half(2 * i2, 0, True, True)
                half(2 * i2 + 1, 1, True, True)
            return 0

        half(0, 0, False, False)
        half(1, 1, False, True)
        lax.fori_loop(1, _CH // 2, body, 0)
        for b in range(2):
            rb, cb = bufs[b]
            pltpu.make_async_copy(rb, outr_hbm.at[pl.ds(0, _C)], wsem).wait()
            pltpu.make_async_copy(cb, outc_hbm.at[pl.ds(0, _C)], wsem).wait()
        pltpu.make_async_copy(db, de_hbm.at[pl.ds(0, _C)], dsem).wait()

    return gather_k


def _make_scatter():
    acc_rows = _NP * 8 // 128

    @functools.partial(
        pl.kernel,
        out_type=jax.ShapeDtypeStruct((_NW, acc_rows, 128), jnp.float32),
        mesh=_sc_mesh(),
        scratch_types=[
            pltpu.VMEM((_CH, _C), jnp.int32),
            pltpu.VMEM((_C, 8), jnp.float32),
            pltpu.VMEM((acc_rows, 128), jnp.float32),
        ],
        compiler_params=pltpu.CompilerParams(needs_layout_passes=False),
    )
    def scatter_k(ve_hbm, col_hbm, zeros_hbm, out_hbm, idxv, vbuf, acc):
        wid = lax.axis_index("s") * _NC + lax.axis_index("c")
        pltpu.sync_copy(zeros_hbm, acc)
        pltpu.sync_copy(col_hbm.at[wid], idxv)
        vbase = wid * _CH * _C
        lane = lax.iota(jnp.int32, 16)

        def body(i, _):
            pltpu.sync_copy(ve_hbm.at[pl.ds(vbase + i * _C, _C)], vbuf)
            for j in range(_C // 16):
                ids = idxv[i, pl.ds(j * 16, 16)]
                rows = j * 16 + lane
                for comp in (0, 1, 2, 4, 5, 6):
                    vals = plsc.load_gather(
                        vbuf, [rows, jnp.full((16,), comp, jnp.int32)])
                    tgt = ids * 8 + comp
                    plsc.addupdate_scatter(
                        acc, [lax.shift_right_logical(tgt, 7),
                              lax.bitwise_and(tgt, 127)], vals)
            return 0

        lax.fori_loop(0, _CH, body, 0)
        pltpu.sync_copy(acc, out_hbm.at[wid])

    return scatter_k


def _edge_body(xr_ref, xc_ref, de_ref, wr_ref, wc_ref, w2_ref, aux_ref,
               out_ref):
    bf = jnp.bfloat16
    f32 = jnp.float32
    xr = xr_ref[...].astype(bf)
    xc = xc_ref[...].astype(bf)
    h = (jnp.dot(xr, wr_ref[...], preferred_element_type=f32)
         + jnp.dot(xc, wc_ref[...], preferred_element_type=f32))
    d = de_ref[...][:, 0:3]
    dist2 = jnp.sum(d * d, axis=1, keepdims=True)
    dist = jnp.sqrt(dist2)
    dterm = (dist.astype(bf).astype(f32)
             * aux_ref[0:1, :].astype(bf).astype(f32))
    h = h + dterm + aux_ref[1:2, :]
    s = h * jax.nn.sigmoid(h)
    mm = jnp.dot(s.astype(bf), w2_ref[...], preferred_element_type=f32)
    m1 = mm[:, 0:1] + aux_ref[3:4, 0:1]
    m2 = mm[:, 1:2] + aux_ref[3:4, 1:2]
    r = dist / 4.5
    r2 = r * r
    r4 = r2 * r2
    r5 = r4 * r
    r6 = r4 * r2
    r7 = r4 * r2 * r
    coe = 1.0 - 21.0 * r5 + 35.0 * r6 - 15.0 * r7
    nv = d / (dist + 1e-8)
    be = xr.shape[0]
    z = jnp.zeros((be, 1), f32)
    vec1 = nv * coe * m1
    vec2 = nv * coe * m2
    out_ref[...] = jnp.concatenate([vec1, z, vec2, z], axis=1)


def _finish_body(p_ref, out_ref):
    v = jnp.sum(p_ref[...], axis=0)
    v1 = v[:, 0:3]
    v2 = v[:, 4:7]
    n1 = v1 / (jnp.sqrt(jnp.sum(v1 * v1, axis=1, keepdims=True)) + 1e-8)
    dot = jnp.sum(n1 * v2, axis=1, keepdims=True)
    n2p = v2 - dot * n1
    n2 = n2p / (jnp.sqrt(jnp.sum(n2p * n2p, axis=1, keepdims=True)) + 1e-8)
    a0, a1, a2 = n1[:, 0:1], n1[:, 1:2], n1[:, 2:3]
    b0, b1, b2 = n2[:, 0:1], n2[:, 1:2], n2[:, 2:3]
    n3 = jnp.concatenate(
        [a1 * b2 - a2 * b1, a2 * b0 - a0 * b2, a0 * b1 - a1 * b0], axis=1)
    out_ref[...] = jnp.concatenate([n1, n2, n3], axis=1)


def kernel(x, pos, W1a, b1a, W1b, b1b, W2a, b2a, W2b, b2b, edge_index):
    N = x.shape[0]
    E = edge_index.shape[1]
    F = x.shape[1]
    H = W1a.shape[1]

    xt = jnp.zeros((_NP, F), jnp.float32).at[:N].set(x)
    pp = jnp.zeros((3, _NP), jnp.float32).at[:, :N].set(pos.T)
    row = (jnp.zeros((_E2,), jnp.int32)
           .at[:E].set(edge_index[0].astype(jnp.int32)).reshape(_NW, _CH, _C))
    col = (jnp.full((_E2,), N, jnp.int32)
           .at[:E].set(edge_index[1].astype(jnp.int32)).reshape(_NW, _CH, _C))

    Wr = jnp.concatenate([W1a[0:F], W2a[0:F]], axis=1).astype(jnp.bfloat16)
    Wc = jnp.concatenate([W1a[F:2 * F], W2a[F:2 * F]],
                         axis=1).astype(jnp.bfloat16)
    W2 = jnp.zeros((2 * H, 128), jnp.float32)
    W2 = W2.at[0:H, 0].set(W1b[:, 0]).at[H:2 * H, 1].set(W2b[:, 0])
    W2 = W2.astype(jnp.bfloat16)

    aux = jnp.zeros((8, 2 * H), jnp.float32)
    aux = aux.at[0, 0:H].set(W1a[2 * F]).at[0, H:].set(W2a[2 * F])
    aux = aux.at[1, 0:H].set(b1a).at[1, H:].set(b2a)
    aux = aux.at[3, 0].set(b1b[0]).at[3, 1].set(b2b[0])

    xer, xec, de = _make_gather(F)(xt, pp[0], pp[1], pp[2], row, col)

    BE = 2048
    ve = pl.pallas_call(
        _edge_body,
        grid=(_E2 // BE,),
        in_specs=[
            pl.BlockSpec((BE, F), lambda i: (i, 0)),
            pl.BlockSpec((BE, F), lambda i: (i, 0)),
            pl.BlockSpec((BE, 8), lambda i: (i, 0)),
            pl.BlockSpec((F, 2 * H), lambda i: (0, 0)),
            pl.BlockSpec((F, 2 * H), lambda i: (0, 0)),
            pl.BlockSpec((2 * H, 128), lambda i: (0, 0)),
            pl.BlockSpec((8, 2 * H), lambda i: (0, 0)),
        ],
        out_specs=pl.BlockSpec((BE, 8), lambda i: (i, 0)),
        out_shape=jax.ShapeDtypeStruct((_E2, 8), jnp.float32),
    )(xer, xec, de, Wr, Wc, W2, aux)

    zacc = jnp.zeros((_NP * 8 // 128, 128), jnp.float32)
    partials = _make_scatter()(ve, col, zacc)
    partials = partials.reshape(_NW, _NP, 8)

    NB = _NP // 8
    out9 = pl.pallas_call(
        _finish_body,
        grid=(_NP // NB,),
        in_specs=[pl.BlockSpec((_NW, NB, 8), lambda i: (0, i, 0))],
        out_specs=pl.BlockSpec((NB, 9), lambda i: (i, 0)),
        out_shape=jax.ShapeDtypeStruct((_NP, 9), jnp.float32),
    )(partials)

    return out9[:N].reshape(N, 3, 3)


# speedup vs baseline: 1.2222x; 1.0056x over previous
import functools

import jax
import jax.numpy as jnp
from jax import lax
from jax.experimental import pallas as pl
from jax.experimental.pallas import tpu as pltpu
from jax.experimental.pallas import tpu_sc as plsc

_NC, _NS = 2, 16
_NW = _NC * _NS
_C = 96
_CH = 106
_E2 = _NW * _CH * _C
_NP = 10240


def _sc_mesh():
    return plsc.VectorSubcoreMesh(
        core_axis_name="c", subcore_axis_name="s",
        num_cores=_NC, num_subcores=_NS)


def _make_gather(F):
    per_w = _CH * _C

    @functools.partial(
        pl.kernel,
        out_type=(jax.ShapeDtypeStruct((_E2, F), jnp.float32),
                  jax.ShapeDtypeStruct((_E2, F), jnp.float32),
                  jax.ShapeDtypeStruct((_E2, 8), jnp.float32)),
        mesh=_sc_mesh(),
        scratch_types=[
            pltpu.VMEM((_CH, _C), jnp.int32),
            pltpu.VMEM((_CH, _C), jnp.int32),
            pltpu.VMEM((_C, F), jnp.float32),
            pltpu.VMEM((_C, F), jnp.float32),
            pltpu.VMEM((_C, F), jnp.float32),
            pltpu.VMEM((_C, F), jnp.float32),
            pltpu.VMEM((_C, 8), jnp.float32),
            pltpu.VMEM((_NP,), jnp.float32),
            pltpu.VMEM((_NP,), jnp.float32),
            pltpu.VMEM((_NP,), jnp.float32),
            pltpu.SemaphoreType.DMA,
            pltpu.SemaphoreType.DMA,
            pltpu.SemaphoreType.DMA,
            pltpu.SemaphoreType.DMA,
        ],
        compiler_params=pltpu.CompilerParams(needs_layout_passes=False),
    )
    def gather_k(x_hbm, px_hbm, py_hbm, pz_hbm, row_hbm, col_hbm,
                 outr_hbm, outc_hbm, de_hbm,
                 idxr, idxc, rb0, cb0, rb1, cb1, db,
                 pxv, pyv, pzv, gsem, gsem2, wsem, dsem):
        wid = lax.axis_index("s") * _NC + lax.axis_index("c")
        base = wid * per_w
        pltpu.sync_copy(row_hbm.at[wid], idxr)
        pltpu.sync_copy(col_hbm.at[wid], idxc)
        pltpu.sync_copy(px_hbm, pxv)
        pltpu.sync_copy(py_hbm, pyv)
        pltpu.sync_copy(pz_hbm, pzv)
        bufs = ((rb0, cb0), (rb1, cb1))
        lane = lax.iota(jnp.int32, 16)

        def half(i, b, drain_rc, drain_d):
            rb, cb = bufs[b]
            if drain_rc:
                pltpu.make_async_copy(rb, outr_hbm.at[pl.ds(0, _C)], wsem).wait()
                pltpu.make_async_copy(cb, outc_hbm.at[pl.ds(0, _C)], wsem).wait()
            gr = pltpu.async_copy(x_hbm.at[idxr.at[i]], rb, gsem)
            gc = pltpu.async_copy(x_hbm.at[idxc.at[i]], cb, gsem2)
            if drain_d:
                pltpu.make_async_copy(db, de_hbm.at[pl.ds(0, _C)], dsem).wait()
            for j in range(_C // 16):
                k16 = j * 16
                ir = idxr[i, pl.ds(k16, 16)]
                ic = idxc[i, pl.ds(k16, 16)]
                ridx = k16 + lane
                for comp, pref in enumerate((pxv, pyv, pzv)):
                    dv = (plsc.load_gather(pref, [ir])
                          - plsc.load_gather(pref, [ic]))
                    cidx = jnp.full((16,), comp, jnp.int32)
                    plsc.store_scatter(db, [ridx, cidx], dv)
            eb = base + i * _C
            pltpu.async_copy(db, de_hbm.at[pl.ds(eb, _C)], dsem)
            gr.wait()
            gc.wait()
            pltpu.async_copy(rb, outr_hbm.at[pl.ds(eb, _C)], wsem)
            pltpu.async_copy(cb, outc_hbm.at[pl.ds(eb, _C)], wsem)

        def body(i2, _):
            @pl.when(i2 > 0)
            def _():
                half(2 * i2, 0, True, True)
                half(2 * i2 + 1, 1, True, True)
            return 0

        half(0, 0, False, False)
        half(1, 1, False, True)
        lax.fori_loop(1, _CH // 2, body, 0)
        for b in range(2):
            rb, cb = bufs[b]
            pltpu.make_async_copy(rb, outr_hbm.at[pl.ds(0, _C)], wsem).wait()
            pltpu.make_async_copy(cb, outc_hbm.at[pl.ds(0, _C)], wsem).wait()
        pltpu.make_async_copy(db, de_hbm.at[pl.ds(0, _C)], dsem).wait()

    return gather_k


def _make_scatter():
    acc_rows = _NP * 8 // 128

    @functools.partial(
        pl.kernel,
        out_type=jax.ShapeDtypeStruct((_NW, acc_rows, 128), jnp.float32),
        mesh=_sc_mesh(),
        scratch_types=[
            pltpu.VMEM((_CH, _C), jnp.int32),
            pltpu.VMEM((_C, 8), jnp.float32),
            pltpu.VMEM((acc_rows, 128), jnp.float32),
        ],
        compiler_params=pltpu.CompilerParams(needs_layout_passes=False),
    )
    def scatter_k(ve_hbm, col_hbm, zeros_hbm, out_hbm, idxv, vbuf, acc):
        wid = lax.axis_index("s") * _NC + lax.axis_index("c")
        pltpu.sync_copy(zeros_hbm, acc)
        pltpu.sync_copy(col_hbm.at[wid], idxv)
        vbase = wid * _CH * _C
        lane = lax.iota(jnp.int32, 16)

        def body(i, _):
            pltpu.sync_copy(ve_hbm.at[pl.ds(vbase + i * _C, _C)], vbuf)
            for j in range(_C // 16):
                ids = idxv[i, pl.ds(j * 16, 16)]
                rows = j * 16 + lane
                for comp in (0, 1, 2, 4, 5, 6):
                    vals = plsc.load_gather(
                        vbuf, [rows, jnp.full((16,), comp, jnp.int32)])
                    tgt = ids * 8 + comp
                    plsc.addupdate_scatter(
                        acc, [lax.shift_right_logical(tgt, 7),
                              lax.bitwise_and(tgt, 127)], vals)
            return 0

        lax.fori_loop(0, _CH, body, 0)
        pltpu.sync_copy(acc, out_hbm.at[wid])

    return scatter_k


def _edge_body(xr_ref, xc_ref, de_ref, wr_ref, wc_ref, w2_ref, aux_ref,
               out_ref):
    bf = jnp.bfloat16
    f32 = jnp.float32
    xr = xr_ref[...].astype(bf)
    xc = xc_ref[...].astype(bf)
    h = (jnp.dot(xr, wr_ref[...], preferred_element_type=f32)
         + jnp.dot(xc, wc_ref[...], preferred_element_type=f32))
    d = de_ref[...][:, 0:3]
    dist2 = jnp.sum(d * d, axis=1, keepdims=True)
    dist = jnp.sqrt(dist2)
    dterm = (dist.astype(bf).astype(f32)
             * aux_ref[0:1, :].astype(bf).astype(f32))
    h = h + dterm + aux_ref[1:2, :]
    s = h * jax.nn.sigmoid(h)
    mm = jnp.dot(s.astype(bf), w2_ref[...], preferred_element_type=f32)
    m1 = mm[:, 0:1] + aux_ref[3:4, 0:1]
    m2 = mm[:, 1:2] + aux_ref[3:4, 1:2]
    r = dist / 4.5
    r2 = r * r
    r4 = r2 * r2
    r5 = r4 * r
    r6 = r4 * r2
    r7 = r4 * r2 * r
    coe = 1.0 - 21.0 * r5 + 35.0 * r6 - 15.0 * r7
    nv = d / (dist + 1e-8)
    be = xr.shape[0]
    z = jnp.zeros((be, 1), f32)
    vec1 = nv * coe * m1
    vec2 = nv * coe * m2
    out_ref[...] = jnp.concatenate([vec1, z, vec2, z], axis=1)


def _finish_body(p_ref, out_ref):
    v = jnp.sum(p_ref[...], axis=0)
    v1 = v[:, 0:3]
    v2 = v[:, 4:7]
    n1 = v1 / (jnp.sqrt(jnp.sum(v1 * v1, axis=1, keepdims=True)) + 1e-8)
    dot = jnp.sum(n1 * v2, axis=1, keepdims=True)
    n2p = v2 - dot * n1
    n2 = n2p / (jnp.sqrt(jnp.sum(n2p * n2p, axis=1, keepdims=True)) + 1e-8)
    a0, a1, a2 = n1[:, 0:1], n1[:, 1:2], n1[:, 2:3]
    b0, b1, b2 = n2[:, 0:1], n2[:, 1:2], n2[:, 2:3]
    n3 = jnp.concatenate(
        [a1 * b2 - a2 * b1, a2 * b0 - a0 * b2, a0 * b1 - a1 * b0], axis=1)
    out_ref[...] = jnp.concatenate([n1, n2, n3], axis=1)


def kernel(x, pos, W1a, b1a, W1b, b1b, W2a, b2a, W2b, b2b, edge_index):
    N = x.shape[0]
    E = edge_index.shape[1]
    F = x.shape[1]
    H = W1a.shape[1]

    xt = jnp.zeros((_NP, F), jnp.float32).at[:N].set(x)
    pp = jnp.zeros((3, _NP), jnp.float32).at[:, :N].set(pos.T)
    row = (jnp.zeros((_E2,), jnp.int32)
           .at[:E].set(edge_index[0].astype(jnp.int32)).reshape(_NW, _CH, _C))
    col = (jnp.full((_E2,), N, jnp.int32)
           .at[:E].set(edge_index[1].astype(jnp.int32)).reshape(_NW, _CH, _C))

    Wr = jnp.concatenate([W1a[0:F], W2a[0:F]], axis=1).astype(jnp.bfloat16)
    Wc = jnp.concatenate([W1a[F:2 * F], W2a[F:2 * F]],
                         axis=1).astype(jnp.bfloat16)
    W2 = jnp.zeros((2 * H, 128), jnp.float32)
    W2 = W2.at[0:H, 0].set(W1b[:, 0]).at[H:2 * H, 1].set(W2b[:, 0])
    W2 = W2.astype(jnp.bfloat16)

    aux = jnp.zeros((8, 2 * H), jnp.float32)
    aux = aux.at[0, 0:H].set(W1a[2 * F]).at[0, H:].set(W2a[2 * F])
    aux = aux.at[1, 0:H].set(b1a).at[1, H:].set(b2a)
    aux = aux.at[3, 0].set(b1b[0]).at[3, 1].set(b2b[0])

    xer, xec, de = _make_gather(F)(xt, pp[0], pp[1], pp[2], row, col)

    BE = 3072
    ve = pl.pallas_call(
        _edge_body,
        grid=(_E2 // BE,),
        in_specs=[
            pl.BlockSpec((BE, F), lambda i: (i, 0)),
            pl.BlockSpec((BE, F), lambda i: (i, 0)),
            pl.BlockSpec((BE, 8), lambda i: (i, 0)),
            pl.BlockSpec((F, 2 * H), lambda i: (0, 0)),
            pl.BlockSpec((F, 2 * H), lambda i: (0, 0)),
            pl.BlockSpec((2 * H, 128), lambda i: (0, 0)),
            pl.BlockSpec((8, 2 * H), lambda i: (0, 0)),
        ],
        out_specs=pl.BlockSpec((BE, 8), lambda i: (i, 0)),
        out_shape=jax.ShapeDtypeStruct((_E2, 8), jnp.float32),
    )(xer, xec, de, Wr, Wc, W2, aux)

    zacc = jnp.zeros((_NP * 8 // 128, 128), jnp.float32)
    partials = _make_scatter()(ve, col, zacc)
    partials = partials.reshape(_NW, _NP, 8)

    NB = _NP // 8
    out9 = pl.pallas_call(
        _finish_body,
        grid=(_NP // NB,),
        in_specs=[pl.BlockSpec((_NW, NB, 8), lambda i: (0, i, 0))],
        out_specs=pl.BlockSpec((NB, 9), lambda i: (i, 0)),
        out_shape=jax.ShapeDtypeStruct((_NP, 9), jnp.float32),
    )(partials)

    return out9[:N].reshape(N, 3, 3)
